# baseline (speedup 1.0000x reference)
import functools

import jax
import jax.numpy as jnp
from jax import lax
from jax.experimental import pallas as pl
from jax.experimental.pallas import tpu as pltpu
from jax.experimental.pallas import tpu_sc as plsc

_N_NODES = 10000
_N_EDGES = 320000
_D = 128
_H = 192
_N_REL = 32

_NUM_CORES = 2
_NUM_SUBCORES = 16
_NW = _NUM_CORES * _NUM_SUBCORES
_PER_W = _N_EDGES // _NW
_CHUNK = 80
_GROUPS = _CHUNK // 16


def _proj_body(x_ref, a_ref, b_ref, hp_ref, tp_ref):
    x = x_ref[...]
    hp_ref[...] = jnp.dot(x, a_ref[...], preferred_element_type=jnp.float32)
    tp_ref[...] = jnp.dot(x, b_ref[...], preferred_element_type=jnp.float32)


def _rq_body(r_ref, c_ref, b1_ref, rq_ref):
    rq_ref[...] = jnp.dot(r_ref[...], c_ref[...],
                          preferred_element_type=jnp.float32) + b1_ref[...]


def _edge_body(hp_hbm, tp_hbm, rq_hbm, w2b_hbm, head_hbm, tail_hbm, typ_hbm,
               noise_hbm, out_hbm,
               idxh_v, idxt_v, typ_v, noise_v, rowsh_v, rowst_v, rq_v, w2b_v,
               out_v, sem1, sem2):
    wid = lax.axis_index("s") * _NUM_CORES + lax.axis_index("c")
    pltpu.sync_copy(rq_hbm, rq_v)
    pltpu.sync_copy(w2b_hbm, w2b_v)
    iota = lax.iota(jnp.int32, 16)

    def chunk_body(i, carry):
        base = wid * _PER_W + i * _CHUNK
        pltpu.sync_copy(head_hbm.at[pl.ds(base, _CHUNK)], idxh_v)
        pltpu.sync_copy(tail_hbm.at[pl.ds(base, _CHUNK)], idxt_v)
        pltpu.sync_copy(typ_hbm.at[pl.ds(base, _CHUNK)], typ_v)
        pltpu.sync_copy(noise_hbm.at[pl.ds(base, _CHUNK)], noise_v)
        c1 = pltpu.async_copy(hp_hbm.at[idxh_v], rowsh_v, sem1)
        c2 = pltpu.async_copy(tp_hbm.at[idxt_v], rowst_v, sem2)
        c1.wait()
        c2.wait()
        tvecs = [typ_v[pl.ds(g * 16, 16)] for g in range(_GROUPS)]

        def d_body(d, accs):
            dsplat = jnp.full((16,), d, dtype=jnp.int32)
            w2v = w2b_v[d, :]
            nxt = []
            for g in range(_GROUPS):
                ev = iota + (g * 16)
                hv = plsc.load_gather(rowsh_v, [ev, dsplat])
                tv = plsc.load_gather(rowst_v, [ev, dsplat])
                rv = plsc.load_gather(rq_v, [tvecs[g], dsplat])
                v = jnp.maximum(hv + tv + rv, 0.0)
                nxt.append(accs[g] + v * w2v)
            return tuple(nxt)

        accs = lax.fori_loop(
            0, _H, d_body,
            tuple(jnp.zeros((16,), jnp.float32) for _ in range(_GROUPS)))
        for g in range(_GROUPS):
            x = accs[g] + noise_v[pl.ds(g * 16, 16)]
            out_v[pl.ds(g * 16, 16)] = 1.0 / (1.0 + jnp.exp(-x))
        pltpu.sync_copy(out_v, out_hbm.at[pl.ds(base, _CHUNK)])
        return carry

    lax.fori_loop(0, _PER_W // _CHUNK, chunk_body, 0)


def kernel(edge_index, edge_type, all_embed, relation_emb, W1, b1, W2, b2):
    temperature = 0.5
    bias = 0.0001

    w1a = W1[:, :_D].T
    w1b = W1[:, _D:2 * _D].T
    w1c = W1[:, 2 * _D:].T
    head = edge_index[0]
    tail = edge_index[1]
    u = jax.random.uniform(jax.random.key(1), (_N_EDGES,), dtype=jnp.float32)
    eps = (bias - (1 - bias)) * u + (1 - bias)
    noise = jnp.log(eps) - jnp.log(1 - eps)
    noise2 = (noise + b2[0]) * (1.0 / temperature)
    w2b = jnp.broadcast_to((W2[0] * (1.0 / temperature))[:, None], (_H, 16))

    nblk = 1000
    hp, tp = pl.pallas_call(
        _proj_body,
        grid=(_N_NODES // nblk,),
        in_specs=[
            pl.BlockSpec((nblk, _D), lambda i: (i, 0)),
            pl.BlockSpec((_D, _H), lambda i: (0, 0)),
            pl.BlockSpec((_D, _H), lambda i: (0, 0)),
        ],
        out_specs=[
            pl.BlockSpec((nblk, _H), lambda i: (i, 0)),
            pl.BlockSpec((nblk, _H), lambda i: (i, 0)),
        ],
        out_shape=[
            jax.ShapeDtypeStruct((_N_NODES, _H), jnp.float32),
            jax.ShapeDtypeStruct((_N_NODES, _H), jnp.float32),
        ],
    )(all_embed, w1a, w1b)

    rq = pl.pallas_call(
        _rq_body,
        out_shape=jax.ShapeDtypeStruct((_N_REL, _H), jnp.float32),
    )(relation_emb, w1c, b1[None, :])

    edge_fn = pl.kernel(
        _edge_body,
        out_type=jax.ShapeDtypeStruct((_N_EDGES,), jnp.float32),
        mesh=plsc.VectorSubcoreMesh(core_axis_name="c", subcore_axis_name="s",
                                    num_cores=_NUM_CORES,
                                    num_subcores=_NUM_SUBCORES),
        compiler_params=pltpu.CompilerParams(use_tc_tiling_on_sc=False,
                                             needs_layout_passes=False),
        scratch_types=[
            pltpu.VMEM((_CHUNK,), jnp.int32),
            pltpu.VMEM((_CHUNK,), jnp.int32),
            pltpu.VMEM((_CHUNK,), jnp.int32),
            pltpu.VMEM((_CHUNK,), jnp.float32),
            pltpu.VMEM((_CHUNK, _H), jnp.float32),
            pltpu.VMEM((_CHUNK, _H), jnp.float32),
            pltpu.VMEM((_N_REL, _H), jnp.float32),
            pltpu.VMEM((_H, 16), jnp.float32),
            pltpu.VMEM((_CHUNK,), jnp.float32),
            pltpu.SemaphoreType.DMA,
            pltpu.SemaphoreType.DMA,
        ],
    )
    return edge_fn(hp, tp, rq, w2b, head, tail, edge_type, noise2)


# speedup vs baseline: 1.1167x; 1.1167x over previous
import functools

import jax
import jax.numpy as jnp
from jax import lax
from jax.experimental import pallas as pl
from jax.experimental.pallas import tpu as pltpu
from jax.experimental.pallas import tpu_sc as plsc

_N_NODES = 10000
_N_EDGES = 320000
_D = 128
_H = 192
_N_REL = 32

_NUM_CORES = 2
_NUM_SUBCORES = 16
_NW = _NUM_CORES * _NUM_SUBCORES
_CHUNK = 128
_GROUPS = _CHUNK // 16
_PER_W = 10240
_N_PAD = _PER_W * _NW
_NCH = _PER_W // _CHUNK


def _proj_body(x_ref, a_ref, b_ref, hp_ref, tp_ref):
    x = x_ref[...]
    hp_ref[...] = jnp.dot(x, a_ref[...], preferred_element_type=jnp.float32)
    tp_ref[...] = jnp.dot(x, b_ref[...], preferred_element_type=jnp.float32)


def _rq_body(r_ref, c_ref, b1_ref, rq_ref):
    rq_ref[...] = jnp.dot(r_ref[...], c_ref[...],
                          preferred_element_type=jnp.float32) + b1_ref[...]


def _edge_body(hp_hbm, tp_hbm, rq_hbm, w2b_hbm, edata_hbm, out_hbm,
               ed0, ed1, rh0, rh1, rt0, rt1, rq_v, w2b_v, out_v,
               sh0, sh1, st0, st1):
    wid = lax.axis_index("s") * _NUM_CORES + lax.axis_index("c")
    pltpu.sync_copy(rq_hbm, rq_v)
    pltpu.sync_copy(w2b_hbm, w2b_v)
    iota = lax.iota(jnp.int32, 16)

    eds = (ed0, ed1)
    rhs = (rh0, rh1)
    rts = (rt0, rt1)
    shs = (sh0, sh1)
    sts = (st0, st1)

    def issue(i, b):
        pltpu.sync_copy(edata_hbm.at[wid * _NCH + i], eds[b])
        pltpu.async_copy(hp_hbm.at[eds[b].at[0]], rhs[b], shs[b])
        pltpu.async_copy(tp_hbm.at[eds[b].at[1]], rts[b], sts[b])

    def wait(b):
        pltpu.make_async_copy(hp_hbm.at[eds[b].at[0]], rhs[b], shs[b]).wait()
        pltpu.make_async_copy(tp_hbm.at[eds[b].at[1]], rts[b], sts[b]).wait()

    def compute(i, b):
        tvecs = [eds[b][2, pl.ds(g * 16, 16)] for g in range(_GROUPS)]
        evs = [iota + (g * 16) for g in range(_GROUPS)]

        def d_body(dd, accs):
            accs = list(accs)
            for k in range(2):
                d = dd * 2 + k
                dsplat = jnp.full((16,), d, dtype=jnp.int32)
                w2v = w2b_v[d, :]
                for g in range(_GROUPS):
                    hv = plsc.load_gather(rhs[b], [evs[g], dsplat])
                    tv = plsc.load_gather(rts[b], [evs[g], dsplat])
                    rv = plsc.load_gather(rq_v, [tvecs[g], dsplat])
                    v = jnp.maximum(hv + tv + rv, 0.0)
                    accs[g] = accs[g] + v * w2v
            return tuple(accs)

        accs = lax.fori_loop(
            0, _H // 2, d_body,
            tuple(jnp.zeros((16,), jnp.float32) for _ in range(_GROUPS)))
        for g in range(_GROUPS):
            noise = plsc.bitcast(eds[b][3, pl.ds(g * 16, 16)], jnp.float32)
            x = accs[g] + noise
            out_v[pl.ds(g * 16, 16)] = 1.0 / (1.0 + jnp.exp(-x))
        pltpu.sync_copy(out_v,
                        out_hbm.at[pl.ds(wid * _PER_W + i * _CHUNK, _CHUNK)])

    issue(0, 0)

    def outer(j, carry):
        for bb in range(2):
            i = j * 2 + bb
            wait(bb)

            @pl.when(i + 1 < _NCH)
            def _():
                issue(i + 1, 1 - bb)

            compute(i, bb)
        return carry

    lax.fori_loop(0, _NCH // 2, outer, 0)


def kernel(edge_index, edge_type, all_embed, relation_emb, W1, b1, W2, b2):
    temperature = 0.5
    bias = 0.0001

    w1a = W1[:, :_D].T
    w1b = W1[:, _D:2 * _D].T
    w1c = W1[:, 2 * _D:].T
    u = jax.random.uniform(jax.random.key(1), (_N_EDGES,), dtype=jnp.float32)
    eps = (bias - (1 - bias)) * u + (1 - bias)
    noise = jnp.log(eps) - jnp.log(1 - eps)
    noise2 = (noise + b2[0]) * (1.0 / temperature)
    w2b = jnp.broadcast_to((W2[0] * (1.0 / temperature))[:, None], (_H, 16))

    pad = _N_PAD - _N_EDGES
    head = jnp.pad(edge_index[0], (0, pad))
    tail = jnp.pad(edge_index[1], (0, pad))
    typ = jnp.pad(edge_type, (0, pad))
    noise_i = jax.lax.bitcast_convert_type(jnp.pad(noise2, (0, pad)),
                                           jnp.int32)
    edata = jnp.stack([head, tail, typ, noise_i])
    edata = edata.reshape(4, _N_PAD // _CHUNK, _CHUNK).transpose(1, 0, 2)

    nblk = 1000
    hp, tp = pl.pallas_call(
        _proj_body,
        grid=(_N_NODES // nblk,),
        in_specs=[
            pl.BlockSpec((nblk, _D), lambda i: (i, 0)),
            pl.BlockSpec((_D, _H), lambda i: (0, 0)),
            pl.BlockSpec((_D, _H), lambda i: (0, 0)),
        ],
        out_specs=[
            pl.BlockSpec((nblk, _H), lambda i: (i, 0)),
            pl.BlockSpec((nblk, _H), lambda i: (i, 0)),
        ],
        out_shape=[
            jax.ShapeDtypeStruct((_N_NODES, _H), jnp.float32),
            jax.ShapeDtypeStruct((_N_NODES, _H), jnp.float32),
        ],
    )(all_embed, w1a, w1b)

    rq = pl.pallas_call(
        _rq_body,
        out_shape=jax.ShapeDtypeStruct((_N_REL, _H), jnp.float32),
    )(relation_emb, w1c, b1[None, :])

    edge_fn = pl.kernel(
        _edge_body,
        out_type=jax.ShapeDtypeStruct((_N_PAD,), jnp.float32),
        mesh=plsc.VectorSubcoreMesh(core_axis_name="c", subcore_axis_name="s",
                                    num_cores=_NUM_CORES,
                                    num_subcores=_NUM_SUBCORES),
        compiler_params=pltpu.CompilerParams(use_tc_tiling_on_sc=False,
                                             needs_layout_passes=False),
        scratch_types=[
            pltpu.VMEM((4, _CHUNK), jnp.int32),
            pltpu.VMEM((4, _CHUNK), jnp.int32),
            pltpu.VMEM((_CHUNK, _H), jnp.float32),
            pltpu.VMEM((_CHUNK, _H), jnp.float32),
            pltpu.VMEM((_CHUNK, _H), jnp.float32),
            pltpu.VMEM((_CHUNK, _H), jnp.float32),
            pltpu.VMEM((_N_REL, _H), jnp.float32),
            pltpu.VMEM((_H, 16), jnp.float32),
            pltpu.VMEM((_CHUNK,), jnp.float32),
            pltpu.SemaphoreType.DMA,
            pltpu.SemaphoreType.DMA,
            pltpu.SemaphoreType.DMA,
            pltpu.SemaphoreType.DMA,
        ],
    )
    out = edge_fn(hp, tp, rq, w2b, edata)
    return out[:_N_EDGES]


# speedup vs baseline: 1.1170x; 1.0003x over previous
import functools

import jax
import jax.numpy as jnp
from jax import lax
from jax.experimental import pallas as pl
from jax.experimental.pallas import tpu as pltpu
from jax.experimental.pallas import tpu_sc as plsc

_N_NODES = 10000
_N_EDGES = 320000
_D = 128
_H = 192
_N_REL = 32

_NUM_CORES = 2
_NUM_SUBCORES = 16
_NW = _NUM_CORES * _NUM_SUBCORES
_CHUNK = 128
_GROUPS = _CHUNK // 16
_PER_W = 10240
_N_PAD = _PER_W * _NW
_NCH = _PER_W // _CHUNK


def _proj_body(x_ref, a_ref, b_ref, hp_ref, tp_ref):
    x = x_ref[...]
    hp_ref[...] = jnp.dot(x, a_ref[...], preferred_element_type=jnp.float32)
    tp_ref[...] = jnp.dot(x, b_ref[...], preferred_element_type=jnp.float32)


def _rq_body(r_ref, c_ref, b1_ref, rq_ref):
    rq_ref[...] = jnp.dot(r_ref[...], c_ref[...],
                          preferred_element_type=jnp.float32) + b1_ref[...]


def _edge_body(hp_hbm, tp_hbm, rq_hbm, w2b_hbm, edata_hbm, out_hbm,
               ed0, ed1, rh0, rh1, rt0, rt1, rq_v, w2b_v, out_v,
               sh0, sh1, st0, st1):
    wid = lax.axis_index("s") * _NUM_CORES + lax.axis_index("c")
    pltpu.sync_copy(rq_hbm, rq_v)
    pltpu.sync_copy(w2b_hbm, w2b_v)
    iota = lax.iota(jnp.int32, 16)

    eds = (ed0, ed1)
    rhs = (rh0, rh1)
    rts = (rt0, rt1)
    shs = (sh0, sh1)
    sts = (st0, st1)

    def issue(i, b):
        pltpu.sync_copy(edata_hbm.at[wid * _NCH + i], eds[b])
        pltpu.async_copy(hp_hbm.at[eds[b].at[0]], rhs[b], shs[b])
        pltpu.async_copy(tp_hbm.at[eds[b].at[1]], rts[b], sts[b])

    def wait(b):
        pltpu.make_async_copy(hp_hbm.at[eds[b].at[0]], rhs[b], shs[b]).wait()
        pltpu.make_async_copy(tp_hbm.at[eds[b].at[1]], rts[b], sts[b]).wait()

    def compute(i, b):
        tvecs = [eds[b][2, pl.ds(g * 16, 16)] for g in range(_GROUPS)]
        evs = [iota + (g * 16) for g in range(_GROUPS)]

        def d_body(dd, accs):
            accs = list(accs)
            for k in range(2):
                d = dd * 2 + k
                dsplat = jnp.full((16,), d, dtype=jnp.int32)
                w2v = w2b_v[d, :]
                for g in range(_GROUPS):
                    hv = plsc.load_gather(rhs[b], [evs[g], dsplat])
                    tv = plsc.load_gather(rts[b], [evs[g], dsplat])
                    rv = plsc.load_gather(rq_v, [tvecs[g], dsplat])
                    v = jnp.maximum(hv + tv + rv, 0.0)
                    accs[g] = accs[g] + v * w2v
            return tuple(accs)

        accs = lax.fori_loop(
            0, _H // 2, d_body,
            tuple(jnp.zeros((16,), jnp.float32) for _ in range(_GROUPS)))
        for g in range(_GROUPS):
            noise = plsc.bitcast(eds[b][3, pl.ds(g * 16, 16)], jnp.float32)
            x = accs[g] + noise
            out_v[pl.ds(g * 16, 16)] = 1.0 / (1.0 + jnp.exp(-x))
        pltpu.sync_copy(out_v,
                        out_hbm.at[pl.ds(wid * _PER_W + i * _CHUNK, _CHUNK)])

    issue(0, 0)

    def outer(j, carry):
        for bb in range(2):
            i = j * 2 + bb
            wait(bb)

            @pl.when(i + 1 < _NCH)
            def _():
                issue(i + 1, 1 - bb)

            compute(i, bb)
        return carry

    lax.fori_loop(0, _NCH // 2, outer, 0)


def kernel(edge_index, edge_type, all_embed, relation_emb, W1, b1, W2, b2):
    temperature = 0.5
    bias = 0.0001

    w1a = W1[:, :_D].T
    w1b = W1[:, _D:2 * _D].T
    w1c = W1[:, 2 * _D:].T
    u = jax.random.uniform(jax.random.key(1), (_N_EDGES,), dtype=jnp.float32)
    eps = (bias - (1 - bias)) * u + (1 - bias)
    noise = jnp.log(eps) - jnp.log(1 - eps)
    noise2 = (noise + b2[0]) * (1.0 / temperature)
    w2b = jnp.broadcast_to((W2[0] * (1.0 / temperature))[:, None], (_H, 16))

    pad = _N_PAD - _N_EDGES
    head = jnp.pad(edge_index[0], (0, pad))
    tail = jnp.pad(edge_index[1], (0, pad))
    typ = jnp.pad(edge_type, (0, pad))
    noise_i = jax.lax.bitcast_convert_type(jnp.pad(noise2, (0, pad)),
                                           jnp.int32)
    edata = jnp.stack([head, tail, typ, noise_i])
    edata = edata.reshape(4, _N_PAD // _CHUNK, _CHUNK).transpose(1, 0, 2)

    nblk = 1000
    hp, tp = pl.pallas_call(
        _proj_body,
        grid=(_N_NODES // nblk,),
        in_specs=[
            pl.BlockSpec((nblk, _D), lambda i: (i, 0)),
            pl.BlockSpec((_D, _H), lambda i: (0, 0)),
            pl.BlockSpec((_D, _H), lambda i: (0, 0)),
        ],
        out_specs=[
            pl.BlockSpec((nblk, _H), lambda i: (i, 0)),
            pl.BlockSpec((nblk, _H), lambda i: (i, 0)),
        ],
        out_shape=[
            jax.ShapeDtypeStruct((_N_NODES, _H), jnp.float32),
            jax.ShapeDtypeStruct((_N_NODES, _H), jnp.float32),
        ],
    )(all_embed, w1a, w1b)

    rq = pl.pallas_call(
        _rq_body,
        out_shape=jax.ShapeDtypeStruct((_N_REL, _H), jnp.float32),
    )(relation_emb, w1c, b1[None, :])

    edge_fn = pl.kernel(
        _edge_body,
        out_type=jax.ShapeDtypeStruct((_N_PAD,), jnp.float32),
        mesh=plsc.VectorSubcoreMesh(core_axis_name="c", subcore_axis_name="s",
                                    num_cores=_NUM_CORES,
                                    num_subcores=_NUM_SUBCORES),
        compiler_params=pltpu.CompilerParams(use_tc_tiling_on_sc=False,
                                             needs_layout_passes=False,
                                             disable_bounds_checks=True),
        scratch_types=[
            pltpu.VMEM((4, _CHUNK), jnp.int32),
            pltpu.VMEM((4, _CHUNK), jnp.int32),
            pltpu.VMEM((_CHUNK, _H), jnp.float32),
            pltpu.VMEM((_CHUNK, _H), jnp.float32),
            pltpu.VMEM((_CHUNK, _H), jnp.float32),
            pltpu.VMEM((_CHUNK, _H), jnp.float32),
            pltpu.VMEM((_N_REL, _H), jnp.float32),
            pltpu.VMEM((_H, 16), jnp.float32),
            pltpu.VMEM((_CHUNK,), jnp.float32),
            pltpu.SemaphoreType.DMA,
            pltpu.SemaphoreType.DMA,
            pltpu.SemaphoreType.DMA,
            pltpu.SemaphoreType.DMA,
        ],
    )
    out = edge_fn(hp, tp, rq, w2b, edata)
    return out[:_N_EDGES]


# speedup vs baseline: 5.1322x; 4.5946x over previous
import functools

import jax
import jax.numpy as jnp
from jax import lax
from jax.experimental import pallas as pl
from jax.experimental.pallas import tpu as pltpu
from jax.experimental.pallas import tpu_sc as plsc

_N_NODES = 10000
_N_EDGES = 320000
_D = 128
_H = 192
_N_REL = 32

_NUM_CORES = 2
_NUM_SUBCORES = 16
_NW = _NUM_CORES * _NUM_SUBCORES
_CHUNK = 128
_GROUPS = _CHUNK // 16
_PER_W = 10240
_N_PAD = _PER_W * _NW
_NCH = _PER_W // _CHUNK


def _proj_body(x_ref, a_ref, b_ref, hp_ref, tp_ref):
    x = x_ref[...]
    hp_ref[...] = jnp.dot(x, a_ref[...], preferred_element_type=jnp.float32)
    tp_ref[...] = jnp.dot(x, b_ref[...], preferred_element_type=jnp.float32)


def _rq_body(r_ref, c_ref, b1_ref, rq_ref):
    rq_ref[...] = jnp.dot(r_ref[...], c_ref[...],
                          preferred_element_type=jnp.float32) + b1_ref[...]


def _edge_body(hp_hbm, tp_hbm, rq_hbm, w2b_hbm, edata_hbm, out_hbm,
               ed0, ed1, rh0, rh1, rt0, rt1, rq_v, w2f_v, sgrp_v, out_v,
               sh0, sh1, st0, st1):
    wid = lax.axis_index("s") * _NUM_CORES + lax.axis_index("c")
    pltpu.sync_copy(rq_hbm, rq_v)
    pltpu.sync_copy(w2b_hbm, w2f_v)
    iota = lax.iota(jnp.int32, 16)

    eds = (ed0, ed1)
    rhs = (rh0, rh1)
    rts = (rt0, rt1)
    shs = (sh0, sh1)
    sts = (st0, st1)

    def issue(i, b):
        pltpu.sync_copy(edata_hbm.at[wid * _NCH + i], eds[b])
        pltpu.async_copy(hp_hbm.at[eds[b].at[0]], rhs[b], shs[b])
        pltpu.async_copy(tp_hbm.at[eds[b].at[1]], rts[b], sts[b])

    def wait(b):
        pltpu.make_async_copy(hp_hbm.at[eds[b].at[0]], rhs[b], shs[b]).wait()
        pltpu.make_async_copy(tp_hbm.at[eds[b].at[1]], rts[b], sts[b]).wait()

    def compute(i, b):
        w2vs = [w2f_v[pl.ds(j * 16, 16)] for j in range(_H // 16)]

        def group_body(g, carry):
            gbase = g * 16
            tvec = eds[b][2, pl.ds(gbase, 16)]
            for m in range(16):
                e = gbase + m
                te = tvec[m]
                p = jnp.zeros((16,), jnp.float32)
                for j in range(_H // 16):
                    hv = rhs[b][e, pl.ds(j * 16, 16)]
                    tv = rts[b][e, pl.ds(j * 16, 16)]
                    rv = rq_v[te, pl.ds(j * 16, 16)]
                    v = jnp.maximum(hv + tv + rv, 0.0)
                    p = p + v * w2vs[j]
                sgrp_v[m, :] = p
            tot = jnp.zeros((16,), jnp.float32)
            for l in range(16):
                col = plsc.load_gather(
                    sgrp_v, [iota, jnp.full((16,), l, dtype=jnp.int32)])
                tot = tot + col
            noise = plsc.bitcast(eds[b][3, pl.ds(gbase, 16)], jnp.float32)
            x = tot + noise
            out_v[pl.ds(gbase, 16)] = 1.0 / (1.0 + jnp.exp(-x))
            return carry

        lax.fori_loop(0, _GROUPS, group_body, 0)
        pltpu.sync_copy(out_v,
                        out_hbm.at[pl.ds(wid * _PER_W + i * _CHUNK, _CHUNK)])

    issue(0, 0)

    def outer(j, carry):
        for bb in range(2):
            i = j * 2 + bb
            wait(bb)

            @pl.when(i + 1 < _NCH)
            def _():
                issue(i + 1, 1 - bb)

            compute(i, bb)
        return carry

    lax.fori_loop(0, _NCH // 2, outer, 0)


def kernel(edge_index, edge_type, all_embed, relation_emb, W1, b1, W2, b2):
    temperature = 0.5
    bias = 0.0001

    w1a = W1[:, :_D].T
    w1b = W1[:, _D:2 * _D].T
    w1c = W1[:, 2 * _D:].T
    u = jax.random.uniform(jax.random.key(1), (_N_EDGES,), dtype=jnp.float32)
    eps = (bias - (1 - bias)) * u + (1 - bias)
    noise = jnp.log(eps) - jnp.log(1 - eps)
    noise2 = (noise + b2[0]) * (1.0 / temperature)
    w2b = W2[0] * (1.0 / temperature)

    pad = _N_PAD - _N_EDGES
    head = jnp.pad(edge_index[0], (0, pad))
    tail = jnp.pad(edge_index[1], (0, pad))
    typ = jnp.pad(edge_type, (0, pad))
    noise_i = jax.lax.bitcast_convert_type(jnp.pad(noise2, (0, pad)),
                                           jnp.int32)
    edata = jnp.stack([head, tail, typ, noise_i])
    edata = edata.reshape(4, _N_PAD // _CHUNK, _CHUNK).transpose(1, 0, 2)

    nblk = 1000
    hp, tp = pl.pallas_call(
        _proj_body,
        grid=(_N_NODES // nblk,),
        in_specs=[
            pl.BlockSpec((nblk, _D), lambda i: (i, 0)),
            pl.BlockSpec((_D, _H), lambda i: (0, 0)),
            pl.BlockSpec((_D, _H), lambda i: (0, 0)),
        ],
        out_specs=[
            pl.BlockSpec((nblk, _H), lambda i: (i, 0)),
            pl.BlockSpec((nblk, _H), lambda i: (i, 0)),
        ],
        out_shape=[
            jax.ShapeDtypeStruct((_N_NODES, _H), jnp.float32),
            jax.ShapeDtypeStruct((_N_NODES, _H), jnp.float32),
        ],
    )(all_embed, w1a, w1b)

    rq = pl.pallas_call(
        _rq_body,
        out_shape=jax.ShapeDtypeStruct((_N_REL, _H), jnp.float32),
    )(relation_emb, w1c, b1[None, :])

    edge_fn = pl.kernel(
        _edge_body,
        out_type=jax.ShapeDtypeStruct((_N_PAD,), jnp.float32),
        mesh=plsc.VectorSubcoreMesh(core_axis_name="c", subcore_axis_name="s",
                                    num_cores=_NUM_CORES,
                                    num_subcores=_NUM_SUBCORES),
        compiler_params=pltpu.CompilerParams(use_tc_tiling_on_sc=False,
                                             needs_layout_passes=False,
                                             disable_bounds_checks=True),
        scratch_types=[
            pltpu.VMEM((4, _CHUNK), jnp.int32),
            pltpu.VMEM((4, _CHUNK), jnp.int32),
            pltpu.VMEM((_CHUNK, _H), jnp.float32),
            pltpu.VMEM((_CHUNK, _H), jnp.float32),
            pltpu.VMEM((_CHUNK, _H), jnp.float32),
            pltpu.VMEM((_CHUNK, _H), jnp.float32),
            pltpu.VMEM((_N_REL, _H), jnp.float32),
            pltpu.VMEM((_H,), jnp.float32),
            pltpu.VMEM((16, 16), jnp.float32),
            pltpu.VMEM((_CHUNK,), jnp.float32),
            pltpu.SemaphoreType.DMA,
            pltpu.SemaphoreType.DMA,
            pltpu.SemaphoreType.DMA,
            pltpu.SemaphoreType.DMA,
        ],
    )
    out = edge_fn(hp, tp, rq, w2b, edata)
    return out[:_N_EDGES]


# speedup vs baseline: 5.2563x; 1.0242x over previous
import functools

import jax
import jax.numpy as jnp
from jax import lax
from jax.experimental import pallas as pl
from jax.experimental.pallas import tpu as pltpu
from jax.experimental.pallas import tpu_sc as plsc

_N_NODES = 10000
_N_EDGES = 320000
_D = 128
_H = 192
_N_REL = 32

_NUM_CORES = 2
_NUM_SUBCORES = 16
_NW = _NUM_CORES * _NUM_SUBCORES
_CHUNK = 64
_GROUPS = _CHUNK // 16
_PER_W = 10240
_N_PAD = _PER_W * _NW
_NCH = _PER_W // _CHUNK
_NBUF = 4


def _proj_body(x_ref, a_ref, b_ref, hp_ref, tp_ref):
    x = x_ref[...]
    hp_ref[...] = jnp.dot(x, a_ref[...], preferred_element_type=jnp.float32)
    tp_ref[...] = jnp.dot(x, b_ref[...], preferred_element_type=jnp.float32)


def _rq_body(r_ref, c_ref, b1_ref, rq_ref):
    rq_ref[...] = jnp.dot(r_ref[...], c_ref[...],
                          preferred_element_type=jnp.float32) + b1_ref[...]


def _edge_body(hp_hbm, tp_hbm, rq_hbm, w2b_hbm, edata_hbm, out_hbm,
               ed0, ed1, ed2, ed3, rh0, rh1, rh2, rh3, rt0, rt1, rt2, rt3,
               rq_v, w2f_v, sgrp_v, out_v,
               se0, se1, se2, se3, sh0, sh1, sh2, sh3, st0, st1, st2, st3):
    wid = lax.axis_index("s") * _NUM_CORES + lax.axis_index("c")
    pltpu.sync_copy(rq_hbm, rq_v)
    pltpu.sync_copy(w2b_hbm, w2f_v)
    iota = lax.iota(jnp.int32, 16)

    eds = (ed0, ed1, ed2, ed3)
    rhs = (rh0, rh1, rh2, rh3)
    rts = (rt0, rt1, rt2, rt3)
    ses = (se0, se1, se2, se3)
    shs = (sh0, sh1, sh2, sh3)
    sts = (st0, st1, st2, st3)

    def issue_meta(i, b):
        pltpu.async_copy(edata_hbm.at[wid * _NCH + i], eds[b], ses[b])

    def wait_meta(b):
        pltpu.make_async_copy(edata_hbm.at[0], eds[b], ses[b]).wait()

    def issue_rows(b):
        pltpu.async_copy(hp_hbm.at[eds[b].at[0]], rhs[b], shs[b])
        pltpu.async_copy(tp_hbm.at[eds[b].at[1]], rts[b], sts[b])

    def wait_rows(b):
        pltpu.make_async_copy(hp_hbm.at[eds[b].at[0]], rhs[b], shs[b]).wait()
        pltpu.make_async_copy(tp_hbm.at[eds[b].at[1]], rts[b], sts[b]).wait()

    def compute(i, b):
        w2vs = [w2f_v[pl.ds(j * 16, 16)] for j in range(_H // 16)]

        def group_body(g, carry):
            gbase = g * 16
            tvec = eds[b][2, pl.ds(gbase, 16)]
            for m in range(16):
                e = gbase + m
                te = tvec[m]
                p = jnp.zeros((16,), jnp.float32)
                for j in range(_H // 16):
                    hv = rhs[b][e, pl.ds(j * 16, 16)]
                    tv = rts[b][e, pl.ds(j * 16, 16)]
                    rv = rq_v[te, pl.ds(j * 16, 16)]
                    v = jnp.maximum(hv + tv + rv, 0.0)
                    p = p + v * w2vs[j]
                sgrp_v[m, :] = p
            tot = jnp.zeros((16,), jnp.float32)
            for l in range(16):
                col = plsc.load_gather(
                    sgrp_v, [iota, jnp.full((16,), l, dtype=jnp.int32)])
                tot = tot + col
            noise = plsc.bitcast(eds[b][3, pl.ds(gbase, 16)], jnp.float32)
            x = tot + noise
            out_v[pl.ds(gbase, 16)] = 1.0 / (1.0 + jnp.exp(-x))
            return carry

        lax.fori_loop(0, _GROUPS, group_body, 0)
        pltpu.sync_copy(out_v,
                        out_hbm.at[pl.ds(wid * _PER_W + i * _CHUNK, _CHUNK)])

    issue_meta(0, 0)
    issue_meta(1, 1)
    issue_meta(2, 2)
    wait_meta(0)
    issue_rows(0)
    wait_meta(1)
    issue_rows(1)

    def outer(j, carry):
        for bb in range(_NBUF):
            i = j * _NBUF + bb
            wait_rows(bb)

            @pl.when(i + 2 < _NCH)
            def _():
                wait_meta((bb + 2) % _NBUF)
                issue_rows((bb + 2) % _NBUF)

            compute(i, bb)

            @pl.when(i + 3 < _NCH)
            def _():
                issue_meta(i + 3, (bb + 3) % _NBUF)

        return carry

    lax.fori_loop(0, _NCH // _NBUF, outer, 0)


def kernel(edge_index, edge_type, all_embed, relation_emb, W1, b1, W2, b2):
    temperature = 0.5
    bias = 0.0001

    w1a = W1[:, :_D].T
    w1b = W1[:, _D:2 * _D].T
    w1c = W1[:, 2 * _D:].T
    u = jax.random.uniform(jax.random.key(1), (_N_EDGES,), dtype=jnp.float32)
    eps = (bias - (1 - bias)) * u + (1 - bias)
    noise = jnp.log(eps) - jnp.log(1 - eps)
    noise2 = (noise + b2[0]) * (1.0 / temperature)
    w2b = W2[0] * (1.0 / temperature)

    pad = _N_PAD - _N_EDGES
    head = jnp.pad(edge_index[0], (0, pad))
    tail = jnp.pad(edge_index[1], (0, pad))
    typ = jnp.pad(edge_type, (0, pad))
    noise_i = jax.lax.bitcast_convert_type(jnp.pad(noise2, (0, pad)),
                                           jnp.int32)
    edata = jnp.stack([head, tail, typ, noise_i])
    edata = edata.reshape(4, _N_PAD // _CHUNK, _CHUNK).transpose(1, 0, 2)

    nblk = 1000
    hp, tp = pl.pallas_call(
        _proj_body,
        grid=(_N_NODES // nblk,),
        in_specs=[
            pl.BlockSpec((nblk, _D), lambda i: (i, 0)),
            pl.BlockSpec((_D, _H), lambda i: (0, 0)),
            pl.BlockSpec((_D, _H), lambda i: (0, 0)),
        ],
        out_specs=[
            pl.BlockSpec((nblk, _H), lambda i: (i, 0)),
            pl.BlockSpec((nblk, _H), lambda i: (i, 0)),
        ],
        out_shape=[
            jax.ShapeDtypeStruct((_N_NODES, _H), jnp.float32),
            jax.ShapeDtypeStruct((_N_NODES, _H), jnp.float32),
        ],
    )(all_embed, w1a, w1b)

    rq = pl.pallas_call(
        _rq_body,
        out_shape=jax.ShapeDtypeStruct((_N_REL, _H), jnp.float32),
    )(relation_emb, w1c, b1[None, :])

    edge_fn = pl.kernel(
        _edge_body,
        out_type=jax.ShapeDtypeStruct((_N_PAD,), jnp.float32),
        mesh=plsc.VectorSubcoreMesh(core_axis_name="c", subcore_axis_name="s",
                                    num_cores=_NUM_CORES,
                                    num_subcores=_NUM_SUBCORES),
        compiler_params=pltpu.CompilerParams(use_tc_tiling_on_sc=False,
                                             needs_layout_passes=False,
                                             disable_bounds_checks=True),
        scratch_types=(
            [pltpu.VMEM((4, _CHUNK), jnp.int32) for _ in range(_NBUF)]
            + [pltpu.VMEM((_CHUNK, _H), jnp.float32) for _ in range(2 * _NBUF)]
            + [
                pltpu.VMEM((_N_REL, _H), jnp.float32),
                pltpu.VMEM((_H,), jnp.float32),
                pltpu.VMEM((16, 16), jnp.float32),
                pltpu.VMEM((_CHUNK,), jnp.float32),
            ]
            + [pltpu.SemaphoreType.DMA for _ in range(3 * _NBUF)]
        ),
    )
    out = edge_fn(hp, tp, rq, w2b, edata)
    return out[:_N_EDGES]


# speedup vs baseline: 6.3688x; 1.2116x over previous
import functools

import jax
import jax.numpy as jnp
from jax import lax
from jax.experimental import pallas as pl
from jax.experimental.pallas import tpu as pltpu
from jax.experimental.pallas import tpu_sc as plsc

_N_NODES = 10000
_N_EDGES = 320000
_D = 128
_H = 192
_N_REL = 32
_HW = _H // 2

_NUM_CORES = 2
_NUM_SUBCORES = 16
_NW = _NUM_CORES * _NUM_SUBCORES
_CHUNK = 128
_GROUPS = _CHUNK // 16
_PER_W = 10240
_N_PAD = _PER_W * _NW
_NCH = _PER_W // _CHUNK
_NBUF = 4


def _proj_body(x_ref, a_ref, b_ref, hp_ref, tp_ref):
    x = x_ref[...]
    hp_ref[...] = jnp.dot(x, a_ref[...], preferred_element_type=jnp.float32)
    tp_ref[...] = jnp.dot(x, b_ref[...], preferred_element_type=jnp.float32)


def _rq_body(r_ref, c_ref, b1_ref, rq_ref):
    rq_ref[...] = jnp.dot(r_ref[...], c_ref[...],
                          preferred_element_type=jnp.float32) + b1_ref[...]


def _edge_body(hp_hbm, tp_hbm, rq_hbm, w2b_hbm, edata_hbm, out_hbm,
               ed0, ed1, ed2, ed3, rh0, rh1, rh2, rh3, rt0, rt1, rt2, rt3,
               rq_v, w2f_v, sgrp_v, out_v,
               se0, se1, se2, se3, sh0, sh1, sh2, sh3, st0, st1, st2, st3):
    wid = lax.axis_index("s") * _NUM_CORES + lax.axis_index("c")
    pltpu.sync_copy(rq_hbm, rq_v)
    pltpu.sync_copy(w2b_hbm, w2f_v)
    iota = lax.iota(jnp.int32, 16)

    eds = (ed0, ed1, ed2, ed3)
    rhs = (rh0, rh1, rh2, rh3)
    rts = (rt0, rt1, rt2, rt3)
    ses = (se0, se1, se2, se3)
    shs = (sh0, sh1, sh2, sh3)
    sts = (st0, st1, st2, st3)

    def issue_meta(i, b):
        pltpu.async_copy(edata_hbm.at[wid * _NCH + i], eds[b], ses[b])

    def wait_meta(b):
        pltpu.make_async_copy(edata_hbm.at[0], eds[b], ses[b]).wait()

    def issue_rows(b):
        pltpu.async_copy(hp_hbm.at[eds[b].at[0]], rhs[b], shs[b])
        pltpu.async_copy(tp_hbm.at[eds[b].at[1]], rts[b], sts[b])

    def wait_rows(b):
        pltpu.make_async_copy(hp_hbm.at[eds[b].at[0]], rhs[b], shs[b]).wait()
        pltpu.make_async_copy(tp_hbm.at[eds[b].at[1]], rts[b], sts[b]).wait()

    def compute(i, b):
        w2es = [w2f_v[pl.ds(j * 16, 16)] for j in range(_HW // 16)]
        w2os = [w2f_v[pl.ds(_HW + j * 16, 16)] for j in range(_HW // 16)]

        def group_body(g, carry):
            gbase = g * 16
            tvec = eds[b][2, pl.ds(gbase, 16)]
            for m in range(16):
                e = gbase + m
                te = tvec[m]
                p = jnp.zeros((16,), jnp.float32)
                for j in range(_HW // 16):
                    hv = plsc.bitcast(rhs[b][e, pl.ds(j * 16, 16)],
                                      jnp.bfloat16)
                    tv = plsc.bitcast(rts[b][e, pl.ds(j * 16, 16)],
                                      jnp.bfloat16)
                    rv = plsc.bitcast(rq_v[te, pl.ds(j * 16, 16)],
                                      jnp.bfloat16)
                    v = jnp.maximum(hv + tv + rv, jnp.bfloat16(0.0))
                    u0, u1 = plsc.unpack(v, format=plsc.PackFormat.INTERLEAVED)
                    p = p + u0 * w2es[j] + u1 * w2os[j]
                sgrp_v[m, :] = p
            tot = jnp.zeros((16,), jnp.float32)
            for l in range(16):
                col = plsc.load_gather(
                    sgrp_v, [iota, jnp.full((16,), l, dtype=jnp.int32)])
                tot = tot + col
            noise = plsc.bitcast(eds[b][3, pl.ds(gbase, 16)], jnp.float32)
            x = tot + noise
            out_v[pl.ds(gbase, 16)] = 1.0 / (1.0 + jnp.exp(-x))
            return carry

        lax.fori_loop(0, _GROUPS, group_body, 0)
        pltpu.sync_copy(out_v,
                        out_hbm.at[pl.ds(wid * _PER_W + i * _CHUNK, _CHUNK)])

    issue_meta(0, 0)
    issue_meta(1, 1)
    issue_meta(2, 2)
    wait_meta(0)
    issue_rows(0)
    wait_meta(1)
    issue_rows(1)

    def outer(j, carry):
        for bb in range(_NBUF):
            i = j * _NBUF + bb
            wait_rows(bb)

            @pl.when(i + 2 < _NCH)
            def _():
                wait_meta((bb + 2) % _NBUF)
                issue_rows((bb + 2) % _NBUF)

            compute(i, bb)

            @pl.when(i + 3 < _NCH)
            def _():
                issue_meta(i + 3, (bb + 3) % _NBUF)

        return carry

    lax.fori_loop(0, _NCH // _NBUF, outer, 0)


def kernel(edge_index, edge_type, all_embed, relation_emb, W1, b1, W2, b2):
    temperature = 0.5
    bias = 0.0001

    w1a = W1[:, :_D].T
    w1b = W1[:, _D:2 * _D].T
    w1c = W1[:, 2 * _D:].T
    u = jax.random.uniform(jax.random.key(1), (_N_EDGES,), dtype=jnp.float32)
    eps = (bias - (1 - bias)) * u + (1 - bias)
    noise = jnp.log(eps) - jnp.log(1 - eps)
    noise2 = (noise + b2[0]) * (1.0 / temperature)
    w2s = W2[0] * (1.0 / temperature)
    w2b = jnp.concatenate([w2s[0::2], w2s[1::2]])

    pad = _N_PAD - _N_EDGES
    head = jnp.pad(edge_index[0], (0, pad))
    tail = jnp.pad(edge_index[1], (0, pad))
    typ = jnp.pad(edge_type, (0, pad))
    noise_i = jax.lax.bitcast_convert_type(jnp.pad(noise2, (0, pad)),
                                           jnp.int32)
    edata = jnp.stack([head, tail, typ, noise_i])
    edata = edata.reshape(4, _N_PAD // _CHUNK, _CHUNK).transpose(1, 0, 2)

    nblk = 1000
    hp, tp = pl.pallas_call(
        _proj_body,
        grid=(_N_NODES // nblk,),
        in_specs=[
            pl.BlockSpec((nblk, _D), lambda i: (i, 0)),
            pl.BlockSpec((_D, _H), lambda i: (0, 0)),
            pl.BlockSpec((_D, _H), lambda i: (0, 0)),
        ],
        out_specs=[
            pl.BlockSpec((nblk, _H), lambda i: (i, 0)),
            pl.BlockSpec((nblk, _H), lambda i: (i, 0)),
        ],
        out_shape=[
            jax.ShapeDtypeStruct((_N_NODES, _H), jnp.float32),
            jax.ShapeDtypeStruct((_N_NODES, _H), jnp.float32),
        ],
    )(all_embed, w1a, w1b)

    rq = pl.pallas_call(
        _rq_body,
        out_shape=jax.ShapeDtypeStruct((_N_REL, _H), jnp.float32),
    )(relation_emb, w1c, b1[None, :])

    def _pack_rows(x):
        xb = x.astype(jnp.bfloat16).reshape(x.shape[0], _HW, 2)
        return jax.lax.bitcast_convert_type(xb, jnp.int32)

    hp = _pack_rows(hp)
    tp = _pack_rows(tp)
    rq = _pack_rows(rq)

    edge_fn = pl.kernel(
        _edge_body,
        out_type=jax.ShapeDtypeStruct((_N_PAD,), jnp.float32),
        mesh=plsc.VectorSubcoreMesh(core_axis_name="c", subcore_axis_name="s",
                                    num_cores=_NUM_CORES,
                                    num_subcores=_NUM_SUBCORES),
        compiler_params=pltpu.CompilerParams(use_tc_tiling_on_sc=False,
                                             needs_layout_passes=False,
                                             disable_bounds_checks=True),
        scratch_types=(
            [pltpu.VMEM((4, _CHUNK), jnp.int32) for _ in range(_NBUF)]
            + [pltpu.VMEM((_CHUNK, _HW), jnp.int32) for _ in range(2 * _NBUF)]
            + [
                pltpu.VMEM((_N_REL, _HW), jnp.int32),
                pltpu.VMEM((_H,), jnp.float32),
                pltpu.VMEM((16, 16), jnp.float32),
                pltpu.VMEM((_CHUNK,), jnp.float32),
            ]
            + [pltpu.SemaphoreType.DMA for _ in range(3 * _NBUF)]
        ),
    )
    out = edge_fn(hp, tp, rq, w2b, edata)
    return out[:_N_EDGES]
